# baseline (speedup 1.0000x reference)
import functools

import jax
import jax.numpy as jnp
from jax import lax
from jax.experimental import pallas as pl
from jax.experimental.pallas import tpu as pltpu
from jax.experimental.pallas import tpu_sc as plsc

N_NODES = 100000
BATCH = 8192
FDIM = 128
KSPLIT = 4096

_NC = 2
_NS = 16
_NW = _NC * _NS

_CHUNK = 128


def _sc_gather_a_body(table_hbm, idxs_hbm, out_hbm, idx_v, rows_v, sem):
    wid = lax.axis_index("s") * _NC + lax.axis_index("c")
    pltpu.sync_copy(idxs_hbm.at[pl.ds(wid, 1)], idx_v)
    pltpu.async_copy(table_hbm.at[idx_v.at[0]], rows_v, sem).wait()
    pltpu.sync_copy(rows_v, out_hbm.at[pl.ds(wid * _CHUNK, _CHUNK)])


@jax.jit
def _sc_gather_a(table, idxs):
    mesh = plsc.VectorSubcoreMesh(core_axis_name="c", subcore_axis_name="s")
    return pl.kernel(
        _sc_gather_a_body,
        out_type=jax.ShapeDtypeStruct((KSPLIT, FDIM), jnp.float32),
        mesh=mesh,
        scratch_types=[
            pltpu.VMEM((1, _CHUNK), jnp.int32),
            pltpu.VMEM((_CHUNK, FDIM), jnp.float32),
            pltpu.SemaphoreType.DMA,
        ],
    )(table, idxs)


def _sc_gather_b_body(table_hbm, idxs_hbm, idxd_hbm, outr_hbm, outd_hbm,
                      idxr_v, idxd_v, rrows_v, drows_v, sem, sem2):
    wid = lax.axis_index("s") * _NC + lax.axis_index("c")
    pltpu.sync_copy(idxs_hbm.at[pl.ds(32 + wid, 1)], idxr_v)
    pltpu.sync_copy(idxd_hbm.at[pl.ds(2 * wid, 2)], idxd_v)
    gathers = [
        pltpu.async_copy(table_hbm.at[idxr_v.at[0]], rrows_v, sem),
        pltpu.async_copy(table_hbm.at[idxd_v.at[0]],
                         drows_v.at[pl.ds(0, _CHUNK)], sem),
        pltpu.async_copy(table_hbm.at[idxd_v.at[1]],
                         drows_v.at[pl.ds(_CHUNK, _CHUNK)], sem),
    ]
    for g in gathers:
        g.wait()
    writes = [
        pltpu.async_copy(rrows_v, outr_hbm.at[pl.ds(wid * _CHUNK, _CHUNK)],
                         sem2),
        pltpu.async_copy(drows_v,
                         outd_hbm.at[pl.ds(wid * 2 * _CHUNK, 2 * _CHUNK)],
                         sem2),
    ]
    for c in writes:
        c.wait()


@jax.jit
def _sc_gather_b(table, idxs, idxd):
    mesh = plsc.VectorSubcoreMesh(core_axis_name="c", subcore_axis_name="s")
    return pl.kernel(
        _sc_gather_b_body,
        out_type=(
            jax.ShapeDtypeStruct((BATCH - KSPLIT, FDIM), jnp.float32),
            jax.ShapeDtypeStruct((BATCH, FDIM), jnp.float32),
        ),
        mesh=mesh,
        scratch_types=[
            pltpu.VMEM((1, _CHUNK), jnp.int32),
            pltpu.VMEM((2, _CHUNK), jnp.int32),
            pltpu.VMEM((_CHUNK, FDIM), jnp.float32),
            pltpu.VMEM((2 * _CHUNK, FDIM), jnp.float32),
            pltpu.SemaphoreType.DMA,
            pltpu.SemaphoreType.DMA,
        ],
    )(table, idxs, idxd)


_BM = 256


def _tc_part1_body(difa_ref, difb_ref, src0_ref, w_ref, out_ref, t_ref):
    m = pl.program_id(0)

    @pl.when(m == 0)
    def _():
        t_ref[...] = jnp.dot(src0_ref[...], w_ref[:FDIM],
                             preferred_element_type=jnp.float32)

    out_ref[:_BM] = jnp.dot(difa_ref[...], t_ref[...],
                            preferred_element_type=jnp.float32)
    out_ref[_BM:] = jnp.dot(difb_ref[...], t_ref[...],
                            preferred_element_type=jnp.float32)


@jax.jit
def _tc_part1(dif_mat, src0, w):
    grid = (BATCH // (2 * _BM),)
    return pl.pallas_call(
        _tc_part1_body,
        grid=grid,
        in_specs=[
            pl.BlockSpec((_BM, KSPLIT), lambda m: (2 * m, 0)),
            pl.BlockSpec((_BM, KSPLIT), lambda m: (2 * m + 1, 0)),
            pl.BlockSpec((KSPLIT, FDIM), lambda m: (0, 0)),
            pl.BlockSpec((2 * FDIM, FDIM), lambda m: (0, 0)),
        ],
        out_specs=pl.BlockSpec((2 * _BM, FDIM), lambda m: (m, 0)),
        out_shape=jax.ShapeDtypeStruct((BATCH, FDIM), jnp.float32),
        scratch_shapes=[
            pltpu.VMEM((KSPLIT, FDIM), jnp.float32),
        ],
        compiler_params=pltpu.CompilerParams(
            dimension_semantics=("arbitrary",),
        ),
    )(dif_mat, dif_mat, src0, w)


def _tc_part2_body(difa_ref, difb_ref, srcr_ref, dst_ref, part_ref, w_ref,
                   out_ref, t_ref):
    m = pl.program_id(0)

    @pl.when(m == 0)
    def _():
        t_ref[...] = jnp.dot(srcr_ref[...], w_ref[:FDIM],
                             preferred_element_type=jnp.float32)

    bias = jnp.dot(dst_ref[...], w_ref[FDIM:],
                   preferred_element_type=jnp.float32)
    agg_a = jnp.dot(difa_ref[...], t_ref[...],
                    preferred_element_type=jnp.float32)
    agg_b = jnp.dot(difb_ref[...], t_ref[...],
                    preferred_element_type=jnp.float32)
    out_ref[:_BM] = jnp.maximum(
        part_ref[:_BM] + agg_a + bias[:_BM], 0.0)
    out_ref[_BM:] = jnp.maximum(
        part_ref[_BM:] + agg_b + bias[_BM:], 0.0)


@jax.jit
def _tc_part2(dif_mat, srcr, dst_rows, partial, w):
    grid = (BATCH // (2 * _BM),)
    return pl.pallas_call(
        _tc_part2_body,
        grid=grid,
        in_specs=[
            pl.BlockSpec((_BM, BATCH - KSPLIT), lambda m: (2 * m, 1)),
            pl.BlockSpec((_BM, BATCH - KSPLIT), lambda m: (2 * m + 1, 1)),
            pl.BlockSpec((BATCH - KSPLIT, FDIM), lambda m: (0, 0)),
            pl.BlockSpec((2 * _BM, FDIM), lambda m: (m, 0)),
            pl.BlockSpec((2 * _BM, FDIM), lambda m: (m, 0)),
            pl.BlockSpec((2 * FDIM, FDIM), lambda m: (0, 0)),
        ],
        out_specs=pl.BlockSpec((2 * _BM, FDIM), lambda m: (m, 0)),
        out_shape=jax.ShapeDtypeStruct((BATCH, FDIM), jnp.float32),
        scratch_shapes=[
            pltpu.VMEM((BATCH - KSPLIT, FDIM), jnp.float32),
        ],
        compiler_params=pltpu.CompilerParams(
            dimension_semantics=("arbitrary",),
        ),
    )(dif_mat, dif_mat, srcr, dst_rows, partial, w)


def kernel(dstsrc_features, dstsrc2src, dstsrc2dst, dif_mat, w):
    idx_src = dstsrc2src.reshape(BATCH // _CHUNK, _CHUNK)
    idx_dst = dstsrc2dst.reshape(BATCH // _CHUNK, _CHUNK)
    src0 = _sc_gather_a(dstsrc_features, idx_src)
    srcr, dst_rows = _sc_gather_b(dstsrc_features, idx_src, idx_dst)
    partial = _tc_part1(dif_mat, src0, w)
    return _tc_part2(dif_mat, srcr, dst_rows, partial, w)


# speedup vs baseline: 1.0297x; 1.0297x over previous
import jax
import jax.numpy as jnp
from jax import lax
from jax.experimental import pallas as pl
from jax.experimental.pallas import tpu as pltpu
from jax.experimental.pallas import tpu_sc as plsc

N_NODES = 100000
BATCH = 8192
FDIM = 128

_NC = 2
_NS = 16
_NW = _NC * _NS

_CHUNK = 128
_HROWS_PER_W = BATCH // _NW
_HNCHUNK = _HROWS_PER_W // _CHUNK


def _sc_gather_body(table_hbm, idxs_hbm, idxd_hbm, outs_hbm, outd_hbm,
                    idxs_v, idxd_v, srows_v, drows_v, sem, sem2):
    wid = lax.axis_index("s") * _NC + lax.axis_index("c")
    base = wid * _HROWS_PER_W
    irow = wid * _HNCHUNK
    pltpu.sync_copy(idxs_hbm.at[pl.ds(irow, _HNCHUNK)], idxs_v)
    pltpu.sync_copy(idxd_hbm.at[pl.ds(irow, _HNCHUNK)], idxd_v)
    gathers = []
    for j in range(_HNCHUNK):
        gathers.append(
            pltpu.async_copy(
                table_hbm.at[idxs_v.at[j]],
                srows_v.at[pl.ds(j * _CHUNK, _CHUNK)],
                sem,
            )
        )
        gathers.append(
            pltpu.async_copy(
                table_hbm.at[idxd_v.at[j]],
                drows_v.at[pl.ds(j * _CHUNK, _CHUNK)],
                sem,
            )
        )
    writes = []
    for j in range(_HNCHUNK):
        gathers[2 * j].wait()
        writes.append(
            pltpu.async_copy(
                srows_v.at[pl.ds(j * _CHUNK, _CHUNK)],
                outs_hbm.at[pl.ds(base + j * _CHUNK, _CHUNK)],
                sem2,
            )
        )
        gathers[2 * j + 1].wait()
        writes.append(
            pltpu.async_copy(
                drows_v.at[pl.ds(j * _CHUNK, _CHUNK)],
                outd_hbm.at[pl.ds(base + j * _CHUNK, _CHUNK)],
                sem2,
            )
        )
    for c in writes:
        c.wait()


@jax.jit
def _sc_gather(table, idx_src, idx_dst):
    mesh = plsc.VectorSubcoreMesh(core_axis_name="c", subcore_axis_name="s")
    return pl.kernel(
        _sc_gather_body,
        out_type=(
            jax.ShapeDtypeStruct((BATCH, FDIM), jnp.float32),
            jax.ShapeDtypeStruct((BATCH, FDIM), jnp.float32),
        ),
        mesh=mesh,
        scratch_types=[
            pltpu.VMEM((_HNCHUNK, _CHUNK), jnp.int32),
            pltpu.VMEM((_HNCHUNK, _CHUNK), jnp.int32),
            pltpu.VMEM((_HROWS_PER_W, FDIM), jnp.float32),
            pltpu.VMEM((_HROWS_PER_W, FDIM), jnp.float32),
            pltpu.SemaphoreType.DMA,
            pltpu.SemaphoreType.DMA,
        ],
    )(table, idx_src, idx_dst)


_BM = 256


def _tc_main_body(difa_ref, difb_ref, src_ref, dst_ref, w_ref, out_ref, t_ref):
    m = pl.program_id(0)

    @pl.when(m == 0)
    def _():
        t_ref[...] = jnp.dot(
            src_ref[...], w_ref[:FDIM],
            preferred_element_type=jnp.float32,
        )

    bias = jnp.dot(dst_ref[...], w_ref[FDIM:],
                   preferred_element_type=jnp.float32)
    agg_a = jnp.dot(difa_ref[...], t_ref[...],
                    preferred_element_type=jnp.float32)
    agg_b = jnp.dot(difb_ref[...], t_ref[...],
                    preferred_element_type=jnp.float32)
    out_ref[:_BM] = jnp.maximum(agg_a + bias[:_BM], 0.0)
    out_ref[_BM:] = jnp.maximum(agg_b + bias[_BM:], 0.0)


@jax.jit
def _tc_main(dif_mat, src_rows, dst_rows, w):
    grid = (BATCH // (2 * _BM),)
    return pl.pallas_call(
        _tc_main_body,
        grid=grid,
        in_specs=[
            pl.BlockSpec((_BM, BATCH), lambda m: (2 * m, 0)),
            pl.BlockSpec((_BM, BATCH), lambda m: (2 * m + 1, 0)),
            pl.BlockSpec((BATCH, FDIM), lambda m: (0, 0)),
            pl.BlockSpec((2 * _BM, FDIM), lambda m: (m, 0)),
            pl.BlockSpec((2 * FDIM, FDIM), lambda m: (0, 0)),
        ],
        out_specs=pl.BlockSpec((2 * _BM, FDIM), lambda m: (m, 0)),
        out_shape=jax.ShapeDtypeStruct((BATCH, FDIM), jnp.float32),
        scratch_shapes=[
            pltpu.VMEM((BATCH, FDIM), jnp.float32),
        ],
        compiler_params=pltpu.CompilerParams(
            dimension_semantics=("arbitrary",),
        ),
    )(dif_mat, dif_mat, src_rows, dst_rows, w)


def kernel(dstsrc_features, dstsrc2src, dstsrc2dst, dif_mat, w):
    idx_src = dstsrc2src.reshape(BATCH // _CHUNK, _CHUNK)
    idx_dst = dstsrc2dst.reshape(BATCH // _CHUNK, _CHUNK)
    src_rows, dst_rows = _sc_gather(dstsrc_features, idx_src, idx_dst)
    return _tc_main(dif_mat, src_rows, dst_rows, w)


# speedup vs baseline: 1.0316x; 1.0019x over previous
import jax
import jax.numpy as jnp
from jax import lax
from jax.experimental import pallas as pl
from jax.experimental.pallas import tpu as pltpu
from jax.experimental.pallas import tpu_sc as plsc

N_NODES = 100000
BATCH = 8192
FDIM = 128

_NC = 2
_NS = 16
_NW = _NC * _NS

_CHUNK = 128
_HROWS_PER_W = BATCH // _NW
_HNCHUNK = _HROWS_PER_W // _CHUNK


def _sc_gather_body(table_hbm, idxs_hbm, idxd_hbm, outs_hbm, outd_hbm,
                    idxs_v, idxd_v, srows_v, drows_v, sem, sem2):
    wid = lax.axis_index("s") * _NC + lax.axis_index("c")
    base = wid * _HROWS_PER_W
    irow = wid * _HNCHUNK
    ic1 = pltpu.async_copy(idxs_hbm.at[pl.ds(irow, _HNCHUNK)], idxs_v, sem2)
    ic2 = pltpu.async_copy(idxd_hbm.at[pl.ds(irow, _HNCHUNK)], idxd_v, sem2)
    ic1.wait()
    ic2.wait()
    gathers = []
    for j in range(_HNCHUNK):
        gathers.append(
            pltpu.async_copy(
                table_hbm.at[idxs_v.at[j]],
                srows_v.at[pl.ds(j * _CHUNK, _CHUNK)],
                sem,
            )
        )
        gathers.append(
            pltpu.async_copy(
                table_hbm.at[idxd_v.at[j]],
                drows_v.at[pl.ds(j * _CHUNK, _CHUNK)],
                sem,
            )
        )
    writes = []
    for j in range(_HNCHUNK):
        gathers[2 * j].wait()
        writes.append(
            pltpu.async_copy(
                srows_v.at[pl.ds(j * _CHUNK, _CHUNK)],
                outs_hbm.at[pl.ds(base + j * _CHUNK, _CHUNK)],
                sem2,
            )
        )
        gathers[2 * j + 1].wait()
        writes.append(
            pltpu.async_copy(
                drows_v.at[pl.ds(j * _CHUNK, _CHUNK)],
                outd_hbm.at[pl.ds(base + j * _CHUNK, _CHUNK)],
                sem2,
            )
        )
    for c in writes:
        c.wait()


@jax.jit
def _sc_gather(table, idx_src, idx_dst):
    mesh = plsc.VectorSubcoreMesh(core_axis_name="c", subcore_axis_name="s")
    return pl.kernel(
        _sc_gather_body,
        out_type=(
            jax.ShapeDtypeStruct((BATCH, FDIM), jnp.float32),
            jax.ShapeDtypeStruct((BATCH, FDIM), jnp.float32),
        ),
        mesh=mesh,
        scratch_types=[
            pltpu.VMEM((_HNCHUNK, _CHUNK), jnp.int32),
            pltpu.VMEM((_HNCHUNK, _CHUNK), jnp.int32),
            pltpu.VMEM((_HROWS_PER_W, FDIM), jnp.float32),
            pltpu.VMEM((_HROWS_PER_W, FDIM), jnp.float32),
            pltpu.SemaphoreType.DMA,
            pltpu.SemaphoreType.DMA,
        ],
    )(table, idx_src, idx_dst)


_BM = 256


def _tc_main_body(difa_ref, difb_ref, src_ref, dst_ref, w_ref, out_ref, t_ref):
    m = pl.program_id(0)

    @pl.when(m == 0)
    def _():
        t_ref[...] = jnp.dot(
            src_ref[...], w_ref[:FDIM],
            preferred_element_type=jnp.float32,
        )

    bias = jnp.dot(dst_ref[...], w_ref[FDIM:],
                   preferred_element_type=jnp.float32)
    agg_a = jnp.dot(difa_ref[...], t_ref[...],
                    preferred_element_type=jnp.float32)
    agg_b = jnp.dot(difb_ref[...], t_ref[...],
                    preferred_element_type=jnp.float32)
    out_ref[:_BM] = jnp.maximum(agg_a + bias[:_BM], 0.0)
    out_ref[_BM:] = jnp.maximum(agg_b + bias[_BM:], 0.0)


@jax.jit
def _tc_main(dif_mat, src_rows, dst_rows, w):
    grid = (BATCH // (2 * _BM),)
    return pl.pallas_call(
        _tc_main_body,
        grid=grid,
        in_specs=[
            pl.BlockSpec((_BM, BATCH), lambda m: (2 * m, 0)),
            pl.BlockSpec((_BM, BATCH), lambda m: (2 * m + 1, 0)),
            pl.BlockSpec((BATCH, FDIM), lambda m: (0, 0)),
            pl.BlockSpec((2 * _BM, FDIM), lambda m: (m, 0)),
            pl.BlockSpec((2 * FDIM, FDIM), lambda m: (0, 0)),
        ],
        out_specs=pl.BlockSpec((2 * _BM, FDIM), lambda m: (m, 0)),
        out_shape=jax.ShapeDtypeStruct((BATCH, FDIM), jnp.float32),
        scratch_shapes=[
            pltpu.VMEM((BATCH, FDIM), jnp.float32),
        ],
        compiler_params=pltpu.CompilerParams(
            dimension_semantics=("arbitrary",),
        ),
    )(dif_mat, dif_mat, src_rows, dst_rows, w)


def kernel(dstsrc_features, dstsrc2src, dstsrc2dst, dif_mat, w):
    idx_src = dstsrc2src.reshape(BATCH // _CHUNK, _CHUNK)
    idx_dst = dstsrc2dst.reshape(BATCH // _CHUNK, _CHUNK)
    src_rows, dst_rows = _sc_gather(dstsrc_features, idx_src, idx_dst)
    return _tc_main(dif_mat, src_rows, dst_rows, w)


# speedup vs baseline: 1.0373x; 1.0055x over previous
import jax
import jax.numpy as jnp
from jax import lax
from jax.experimental import pallas as pl
from jax.experimental.pallas import tpu as pltpu
from jax.experimental.pallas import tpu_sc as plsc

N_NODES = 100000
BATCH = 8192
FDIM = 128

_NC = 2
_NS = 16
_NW = _NC * _NS

_CHUNK = 128
_HROWS_PER_W = BATCH // _NW
_HNCHUNK = _HROWS_PER_W // _CHUNK


def _sc_gather_body(table_hbm, idxs_hbm, idxd_hbm, outs_hbm, outd_hbm,
                    idxs_v, idxd_v, srows_v, drows_v, sem, sem2):
    wid = lax.axis_index("s") * _NC + lax.axis_index("c")
    base = wid * _HROWS_PER_W
    irow = wid * _HNCHUNK
    ic1 = pltpu.async_copy(idxs_hbm.at[pl.ds(irow, _HNCHUNK)], idxs_v, sem2)
    ic2 = pltpu.async_copy(idxd_hbm.at[pl.ds(irow, _HNCHUNK)], idxd_v, sem2)
    ic1.wait()
    ic2.wait()
    gathers = []
    for j in range(_HNCHUNK):
        gathers.append(
            pltpu.async_copy(
                table_hbm.at[idxs_v.at[j]],
                srows_v.at[pl.ds(j * _CHUNK, _CHUNK)],
                sem,
            )
        )
        gathers.append(
            pltpu.async_copy(
                table_hbm.at[idxd_v.at[j]],
                drows_v.at[pl.ds(j * _CHUNK, _CHUNK)],
                sem,
            )
        )
    writes = []
    for j in range(_HNCHUNK):
        gathers[2 * j].wait()
        writes.append(
            pltpu.async_copy(
                srows_v.at[pl.ds(j * _CHUNK, _CHUNK)],
                outs_hbm.at[pl.ds(base + j * _CHUNK, _CHUNK)],
                sem2,
            )
        )
        gathers[2 * j + 1].wait()
        writes.append(
            pltpu.async_copy(
                drows_v.at[pl.ds(j * _CHUNK, _CHUNK)],
                outd_hbm.at[pl.ds(base + j * _CHUNK, _CHUNK)],
                sem2,
            )
        )
    for c in writes:
        c.wait()


@jax.jit
def _sc_gather(table, idx_src, idx_dst):
    mesh = plsc.VectorSubcoreMesh(core_axis_name="c", subcore_axis_name="s")
    return pl.kernel(
        _sc_gather_body,
        out_type=(
            jax.ShapeDtypeStruct((BATCH, FDIM), jnp.float32),
            jax.ShapeDtypeStruct((BATCH, FDIM), jnp.float32),
        ),
        mesh=mesh,
        scratch_types=[
            pltpu.VMEM((_HNCHUNK, _CHUNK), jnp.int32),
            pltpu.VMEM((_HNCHUNK, _CHUNK), jnp.int32),
            pltpu.VMEM((_HROWS_PER_W, FDIM), jnp.float32),
            pltpu.VMEM((_HROWS_PER_W, FDIM), jnp.float32),
            pltpu.SemaphoreType.DMA,
            pltpu.SemaphoreType.DMA,
        ],
    )(table, idx_src, idx_dst)


_BM = 256


_BQ = 128


def _tc_main_body(difa_ref, difb_ref, difc_ref, difd_ref,
                  src_ref, dst_ref, w_ref, out_ref, t_ref):
    m = pl.program_id(0)

    @pl.when(m == 0)
    def _():
        t_ref[...] = jnp.dot(
            src_ref[...], w_ref[:FDIM],
            preferred_element_type=jnp.float32,
        )

    bias = jnp.dot(dst_ref[...], w_ref[FDIM:],
                   preferred_element_type=jnp.float32)
    for i, ref in enumerate((difa_ref, difb_ref, difc_ref, difd_ref)):
        agg = jnp.dot(ref[...], t_ref[...],
                      preferred_element_type=jnp.float32)
        out_ref[i * _BQ:(i + 1) * _BQ] = jnp.maximum(
            agg + bias[i * _BQ:(i + 1) * _BQ], 0.0)


@jax.jit
def _tc_main(dif_mat, src_rows, dst_rows, w):
    grid = (BATCH // (4 * _BQ),)
    return pl.pallas_call(
        _tc_main_body,
        grid=grid,
        in_specs=[
            pl.BlockSpec((_BQ, BATCH), lambda m: (4 * m, 0)),
            pl.BlockSpec((_BQ, BATCH), lambda m: (4 * m + 1, 0)),
            pl.BlockSpec((_BQ, BATCH), lambda m: (4 * m + 2, 0)),
            pl.BlockSpec((_BQ, BATCH), lambda m: (4 * m + 3, 0)),
            pl.BlockSpec((BATCH, FDIM), lambda m: (0, 0)),
            pl.BlockSpec((4 * _BQ, FDIM), lambda m: (m, 0)),
            pl.BlockSpec((2 * FDIM, FDIM), lambda m: (0, 0)),
        ],
        out_specs=pl.BlockSpec((4 * _BQ, FDIM), lambda m: (m, 0)),
        out_shape=jax.ShapeDtypeStruct((BATCH, FDIM), jnp.float32),
        scratch_shapes=[
            pltpu.VMEM((BATCH, FDIM), jnp.float32),
        ],
        compiler_params=pltpu.CompilerParams(
            dimension_semantics=("arbitrary",),
        ),
    )(dif_mat, dif_mat, dif_mat, dif_mat, src_rows, dst_rows, w)


def kernel(dstsrc_features, dstsrc2src, dstsrc2dst, dif_mat, w):
    idx_src = dstsrc2src.reshape(BATCH // _CHUNK, _CHUNK)
    idx_dst = dstsrc2dst.reshape(BATCH // _CHUNK, _CHUNK)
    src_rows, dst_rows = _sc_gather(dstsrc_features, idx_src, idx_dst)
    return _tc_main(dif_mat, src_rows, dst_rows, w)
